# baseline (speedup 1.0000x reference)
import functools
import jax
import jax.numpy as jnp
from jax import lax
from jax.experimental import pallas as pl
from jax.experimental.pallas import tpu as pltpu
from jax.experimental.pallas import tpu_sc as plsc


def _ka(adj_ref, wmlp_ref, bmlp_ref, x_ref, wc0_ref, wc1_ref, wc2_ref,
        wc3_ref, we1_ref, bc0_ref,
        ssoft_ref, sq_ref, sqt_ref, dg_ref, diag_ref, z0_ref, h1_ref,
        h2_ref, h3_ref, hs_ref, sts_ref, fro2_ref, *, R, N, C):
    pi = pl.program_id(0)
    ab = adj_ref[...]
    rid = jax.lax.broadcasted_iota(jnp.int32, (R, N), 0) + pi * R
    cid = jax.lax.broadcasted_iota(jnp.int32, (R, N), 1)
    dmask = rid == cid
    diag = jnp.sum(jnp.where(dmask, ab, 0.0), axis=1, keepdims=True)

    sraw_a = jnp.dot(ab, wmlp_ref[...], preferred_element_type=jnp.float32)
    rowsum = sraw_a[:, C:C + 1]
    sraw = sraw_a[:, :C] + bmlp_ref[...]

    deg = jnp.maximum(rowsum - diag + 1.0, 1.0)
    dg = jax.lax.rsqrt(deg)
    dg_ref[...] = dg
    diag_ref[...] = diag

    m = jnp.max(sraw, axis=1, keepdims=True)
    e = jnp.exp(sraw - m)
    ssoft = e / jnp.sum(e, axis=1, keepdims=True)
    ssoft_ref[...] = ssoft
    sq = jnp.sum(ssoft * ssoft, axis=1, keepdims=True)
    sq_ref[...] = sq
    sqt_ref[...] = sq.T

    xb = x_ref[...]
    z0_ref[...] = jnp.maximum(
        jnp.dot(xb, wc0_ref[...], preferred_element_type=jnp.float32)
        + bc0_ref[...], 0.0)
    h1_ref[...] = jnp.dot(xb, wc1_ref[...], preferred_element_type=jnp.float32)
    h2_ref[...] = jnp.dot(xb, wc2_ref[...], preferred_element_type=jnp.float32)
    h3_ref[...] = jnp.dot(xb, wc3_ref[...], preferred_element_type=jnp.float32)
    he1 = jnp.dot(xb, we1_ref[...], preferred_element_type=jnp.float32)
    hs_ref[...] = jnp.concatenate([dg * he1, ssoft], axis=1)

    stsb = jax.lax.dot_general(ssoft, ssoft, (((0,), (0,)), ((), ())),
                               preferred_element_type=jnp.float32)
    frob = jnp.sum(ab * ab).reshape(1, 1)

    @pl.when(pi == 0)
    def _():
        sts_ref[...] = stsb
        fro2_ref[...] = frob

    @pl.when(pi > 0)
    def _():
        sts_ref[...] += stsb
        fro2_ref[...] += frob


def _kc(adj_ref, ssoft_ref, sq_ref, sqt_ref, dg_ref, diag_ref, hs_ref,
        be1_ref, h1_ref, h2_ref, h3_ref, we2_ref,
        hd2_ref, t1_ref, t2_ref, t3_ref, i1_ref, i2_ref, i3_ref,
        w1_ref, w2_ref, w3_ref, cross_ref, *, R, N, H):
    pi = pl.program_id(0)
    ab = adj_ref[...]
    ssoft_f = ssoft_ref[...]
    ssoft_b = ssoft_ref[pl.ds(pi * R, R), :]
    sq_b = sq_ref[pl.ds(pi * R, R), :]
    sq_row = sqt_ref[...]

    ssm = jax.lax.dot_general(-2.0 * ssoft_b, ssoft_f,
                              (((1,), (1,)), ((), ())),
                              preferred_element_type=jnp.float32)

    big2 = jnp.dot(ab, hs_ref[...], preferred_element_type=jnp.float32)
    acc = big2[:, :H]
    p = big2[:, H:]
    cb = jnp.sum(ssoft_b * p).reshape(1, 1)

    @pl.when(pi == 0)
    def _():
        cross_ref[...] = cb

    @pl.when(pi > 0)
    def _():
        cross_ref[...] += cb

    d2 = jnp.maximum(sq_b + sq_row + ssm, 1e-9)

    cidf = jax.lax.broadcasted_iota(jnp.int32, (R, N), 1).astype(jnp.float32)
    bigi = jnp.float32(N)
    big = jnp.float32(jnp.inf)

    m1 = jnp.min(d2, axis=1, keepdims=True)
    i1f = jnp.min(jnp.where(d2 == m1, cidf, bigi), axis=1, keepdims=True)
    d2 = jnp.where(cidf == i1f, big, d2)
    m2 = jnp.min(d2, axis=1, keepdims=True)
    i2f = jnp.min(jnp.where(d2 == m2, cidf, bigi), axis=1, keepdims=True)
    d2 = jnp.where(cidf == i2f, big, d2)
    m3 = jnp.min(d2, axis=1, keepdims=True)
    i3f = jnp.min(jnp.where(d2 == m3, cidf, bigi), axis=1, keepdims=True)

    i1 = i1f.astype(jnp.int32)
    i2 = i2f.astype(jnp.int32)
    i3 = i3f.astype(jnp.int32)
    i1_ref[...] = i1
    i2_ref[...] = i2
    i3_ref[...] = i3
    v1 = jnp.exp(-jnp.sqrt(m1))
    v2 = jnp.exp(-jnp.sqrt(m2))
    v3 = jnp.exp(-jnp.sqrt(m3))
    w1_ref[...] = v1
    w2_ref[...] = v2
    w3_ref[...] = v3

    rid_b = jax.lax.broadcasted_iota(jnp.int32, (R, 1), 0) + pi * R
    for i_b, v_b, h_ref, t_ref in ((i1, v1, h1_ref, t1_ref),
                                   (i2, v2, h2_ref, t2_ref),
                                   (i3, v3, h3_ref, t3_ref)):
        off = i_b != rid_b
        dgj = jax.lax.rsqrt(1.0 + jnp.where(off, v_b, 0.0))
        t_ref[...] = dgj * h_ref[...]

    dg_b = dg_ref[pl.ds(pi * R, R), :]
    hd1_blk = hs_ref[pl.ds(pi * R, R), :H]
    acc = acc + (1.0 - diag_ref[...]) * hd1_blk
    extra1 = jnp.maximum(dg_b * acc + be1_ref[...], 0.0)
    hd2_ref[...] = dg_b * jnp.dot(extra1, we2_ref[...],
                                  preferred_element_type=jnp.float32)


def _sc_gather(t1, t2, t3, idx1, idx2, idx3):
    N, H = t1.shape
    info = plsc.get_sparse_core_info()
    nw = info.num_cores * info.num_subcores
    bpw = N // nw
    mesh = plsc.VectorSubcoreMesh(core_axis_name="c", subcore_axis_name="s")
    f32 = jnp.float32

    @functools.partial(
        pl.kernel, mesh=mesh,
        out_type=[jax.ShapeDtypeStruct((N, H), f32)] * 3,
        scratch_types=[
            pltpu.VMEM((bpw,), jnp.int32),
            pltpu.VMEM((bpw, H), f32),
            pltpu.SemaphoreType.DMA,
        ],
    )
    def g(t1_hbm, t2_hbm, t3_hbm, i1_hbm, i2_hbm, i3_hbm,
          o1_hbm, o2_hbm, o3_hbm, idx_v, rows_v, sem):
        wid = lax.axis_index("s") * info.num_cores + lax.axis_index("c")
        base = wid * bpw
        for t_hbm, i_hbm, o_hbm in ((t1_hbm, i1_hbm, o1_hbm),
                                    (t2_hbm, i2_hbm, o2_hbm),
                                    (t3_hbm, i3_hbm, o3_hbm)):
            pltpu.sync_copy(i_hbm.at[pl.ds(base, bpw)], idx_v)
            pltpu.async_copy(t_hbm.at[idx_v], rows_v, sem).wait()
            pltpu.sync_copy(rows_v, o_hbm.at[pl.ds(base, bpw)])

    return g(t1, t2, t3, idx1, idx2, idx3)


def _kd1(adj_ref, hd2_ref, dg_ref, diag_ref, be2_ref, extra2_ref, *, R, N):
    pi = pl.program_id(0)
    ab = adj_ref[...]
    hd2 = hd2_ref[...]
    dg_b = dg_ref[pl.ds(pi * R, R), :]
    hd2_b = hd2_ref[pl.ds(pi * R, R), :]
    acc = jnp.dot(ab, hd2, preferred_element_type=jnp.float32)
    acc = acc + (1.0 - diag_ref[...]) * hd2_b
    extra2_ref[...] = jnp.maximum(dg_b * acc + be2_ref[...], 0.0)


def _kd2(extra2_ref, z0_ref,
         t1_ref, t2_ref, t3_ref, g1_ref, g2_ref, g3_ref,
         i1_ref, i2_ref, i3_ref, w1_ref, w2_ref, w3_ref,
         bc1_ref, bc2_ref, bc3_ref, wcls_ref, bcls_ref, sts_ref,
         cross_ref, fro2_ref, out_ref, loss_ref, *, R, N, K):
    pi = pl.program_id(0)
    extra2 = extra2_ref[...]

    rid_b = jax.lax.broadcasted_iota(jnp.int32, (R, 1), 0) + pi * R

    def jump(t_ref, g_ref, i_ref, w_ref, bc_ref):
        idx_b = i_ref[...]
        v_b = w_ref[...]
        coef = jnp.where(idx_b != rid_b, v_b, 0.0)
        dgj = jax.lax.rsqrt(1.0 + coef)
        return jnp.maximum(
            dgj * (t_ref[...] + coef * g_ref[...]) + bc_ref[...], 0.0)

    z1 = jump(t1_ref, g1_ref, i1_ref, w1_ref, bc1_ref)
    z2 = jump(t2_ref, g2_ref, i2_ref, w2_ref, bc2_ref)
    z3 = jump(t3_ref, g3_ref, i3_ref, w3_ref, bc3_ref)

    feat = jnp.concatenate([z0_ref[...], z1, z2, z3, extra2], axis=1)
    u = jnp.dot(feat, wcls_ref[...], preferred_element_type=jnp.float32)
    u = u + bcls_ref[...]
    mx = jnp.max(u, axis=1, keepdims=True)
    um = u - mx
    lse = jnp.log(jnp.sum(jnp.exp(um), axis=1, keepdims=True))
    out_ref[...] = um - lse

    @pl.when(pi == 0)
    def _():
        sts = sts_ref[...]
        nsts2 = jnp.sum(sts * sts)
        fro2 = fro2_ref[0, 0]
        resid = jnp.maximum(fro2 - 2.0 * cross_ref[0, 0] + nsts2, 0.0)
        pump = jnp.sqrt(resid) * jax.lax.rsqrt(fro2)
        nsts = jnp.sqrt(nsts2)
        rid = jax.lax.broadcasted_iota(jnp.int32, (K, K), 0)
        cid = jax.lax.broadcasted_iota(jnp.int32, (K, K), 1)
        eye = jnp.where(rid == cid, 1.0, 0.0)
        t = sts / nsts - eye * jax.lax.rsqrt(jnp.float32(K))
        ortho = jnp.sqrt(jnp.sum(t * t))
        loss_ref[...] = (pump + ortho).reshape(1, 1)


def kernel(x, adj, W_mlp, b_mlp, Wc, bc, We1, be1, We2, be2, att, Wcls,
           bcls):
    N = adj.shape[0]
    F = x.shape[1]
    C = W_mlp.shape[1]
    H = Wc.shape[2]
    O = Wcls.shape[1]
    R = 512 if N % 512 == 0 else N
    NB = N // R
    f32 = jnp.float32
    i32 = jnp.int32

    mask_att = jax.nn.softmax(att, axis=0)
    wc0 = Wc[0] * mask_att[0]
    bc0 = (bc[0] * mask_att[0]).reshape(1, H)
    bc1 = (bc[1] * mask_att[1]).reshape(1, H)
    bc2 = (bc[2] * mask_att[2]).reshape(1, H)
    bc3 = (bc[3] * mask_att[3]).reshape(1, H)
    wc1m, wc2m, wc3m = (Wc[1] * mask_att[1], Wc[2] * mask_att[2],
                        Wc[3] * mask_att[3])
    we2m = We2 * mask_att[4]
    be2m = (be2 * mask_att[4]).reshape(1, H)
    bmlp = b_mlp.reshape(1, C)
    be1r = be1.reshape(1, H)
    bclsr = bcls.reshape(1, O)

    row_blk = lambda w: pl.BlockSpec((R, w), lambda i: (i, 0))
    full = lambda a, b: pl.BlockSpec((a, b), lambda i: (0, 0))
    col_blk = pl.BlockSpec((1, R), lambda i: (0, i))
    seq = pltpu.CompilerParams(dimension_semantics=("arbitrary",))

    wmlp_aug = jnp.concatenate([W_mlp, jnp.ones((N, 1), f32)], axis=1)

    (ssoft, sq, sqt, dg, diag, z0, h1, h2, h3, hs, sts,
     fro2) = pl.pallas_call(
        functools.partial(_ka, R=R, N=N, C=C),
        grid=(NB,),
        in_specs=[row_blk(N), full(N, C + 1), full(1, C), row_blk(F),
                  full(F, H), full(F, H), full(F, H), full(F, H),
                  full(F, H), full(1, H)],
        out_specs=[row_blk(C), row_blk(1), col_blk, row_blk(1),
                   row_blk(1), row_blk(H), row_blk(H), row_blk(H),
                   row_blk(H), row_blk(2 * H), full(C, C), full(1, 1)],
        out_shape=[
            jax.ShapeDtypeStruct((N, C), f32),
            jax.ShapeDtypeStruct((N, 1), f32),
            jax.ShapeDtypeStruct((1, N), f32),
            jax.ShapeDtypeStruct((N, 1), f32),
            jax.ShapeDtypeStruct((N, 1), f32),
            jax.ShapeDtypeStruct((N, H), f32),
            jax.ShapeDtypeStruct((N, H), f32),
            jax.ShapeDtypeStruct((N, H), f32),
            jax.ShapeDtypeStruct((N, H), f32),
            jax.ShapeDtypeStruct((N, 2 * H), f32),
            jax.ShapeDtypeStruct((C, C), f32),
            jax.ShapeDtypeStruct((1, 1), f32),
        ],
        compiler_params=seq,
    )(adj, wmlp_aug, bmlp, x, wc0, wc1m, wc2m, wc3m, We1, bc0)

    hd2, t1, t2, t3, i1, i2, i3, w1, w2, w3, cross = pl.pallas_call(
        functools.partial(_kc, R=R, N=N, H=H),
        grid=(NB,),
        in_specs=[row_blk(N), full(N, C), full(N, 1), full(1, N),
                  full(N, 1), row_blk(1), full(N, 2 * H), full(1, H),
                  row_blk(H), row_blk(H), row_blk(H), full(H, H)],
        out_specs=[row_blk(H), row_blk(H), row_blk(H), row_blk(H),
                   row_blk(1), row_blk(1), row_blk(1), row_blk(1),
                   row_blk(1), row_blk(1), full(1, 1)],
        out_shape=[
            jax.ShapeDtypeStruct((N, H), f32),
            jax.ShapeDtypeStruct((N, H), f32),
            jax.ShapeDtypeStruct((N, H), f32),
            jax.ShapeDtypeStruct((N, H), f32),
            jax.ShapeDtypeStruct((N, 1), i32),
            jax.ShapeDtypeStruct((N, 1), i32),
            jax.ShapeDtypeStruct((N, 1), i32),
            jax.ShapeDtypeStruct((N, 1), f32),
            jax.ShapeDtypeStruct((N, 1), f32),
            jax.ShapeDtypeStruct((N, 1), f32),
            jax.ShapeDtypeStruct((1, 1), f32),
        ],
        compiler_params=seq,
    )(adj, ssoft, sq, sqt, dg, diag, hs, be1r, h1, h2, h3, we2m)

    g1, g2, g3 = _sc_gather(t1, t2, t3,
                            i1.reshape(N), i2.reshape(N), i3.reshape(N))

    extra2 = pl.pallas_call(
        functools.partial(_kd1, R=R, N=N),
        grid=(NB,),
        in_specs=[row_blk(N), full(N, H), full(N, 1), row_blk(1),
                  full(1, H)],
        out_specs=[row_blk(H)],
        out_shape=[jax.ShapeDtypeStruct((N, H), f32)],
        compiler_params=seq,
    )(adj, hd2, dg, diag, be2m)[0]

    out, loss = pl.pallas_call(
        functools.partial(_kd2, R=R, N=N, K=C),
        grid=(NB,),
        in_specs=[row_blk(H), row_blk(H), row_blk(H), row_blk(H),
                  row_blk(H), row_blk(H), row_blk(H), row_blk(H),
                  row_blk(1), row_blk(1), row_blk(1), row_blk(1),
                  row_blk(1), row_blk(1), full(1, H), full(1, H),
                  full(1, H), full(5 * H, O), full(1, O), full(C, C),
                  full(1, 1), full(1, 1)],
        out_specs=[row_blk(O), full(1, 1)],
        out_shape=[
            jax.ShapeDtypeStruct((N, O), f32),
            jax.ShapeDtypeStruct((1, 1), f32),
        ],
        compiler_params=seq,
    )(extra2, z0, t1, t2, t3, g1, g2, g3, i1, i2, i3,
      w1, w2, w3, bc1, bc2, bc3, Wcls, bclsr, sts, cross, fro2)

    return out, loss[0, 0]


# speedup vs baseline: 1.0102x; 1.0102x over previous
import functools
import jax
import jax.numpy as jnp
from jax import lax
from jax.experimental import pallas as pl
from jax.experimental.pallas import tpu as pltpu
from jax.experimental.pallas import tpu_sc as plsc


def _ka(adj_ref, wmlp_ref, bmlp_ref, x_ref, wc0_ref, wc1_ref, wc2_ref,
        wc3_ref, we1_ref, bc0_ref,
        ssoft_ref, sq_ref, sqt_ref, dg_ref, diag_ref, z0_ref, h1_ref,
        h2_ref, h3_ref, hs_ref, sts_ref, fro2_ref, *, R, N, C):
    pi = pl.program_id(0)
    ab = adj_ref[...]
    rid = jax.lax.broadcasted_iota(jnp.int32, (R, N), 0) + pi * R
    cid = jax.lax.broadcasted_iota(jnp.int32, (R, N), 1)
    dmask = rid == cid
    diag = jnp.sum(jnp.where(dmask, ab, 0.0), axis=1, keepdims=True)

    sraw_a = jnp.dot(ab, wmlp_ref[...], preferred_element_type=jnp.float32)
    rowsum = sraw_a[:, C:C + 1]
    sraw = sraw_a[:, :C] + bmlp_ref[...]

    deg = jnp.maximum(rowsum - diag + 1.0, 1.0)
    dg = jax.lax.rsqrt(deg)
    dg_ref[...] = dg
    diag_ref[...] = diag

    m = jnp.max(sraw, axis=1, keepdims=True)
    e = jnp.exp(sraw - m)
    ssoft = e / jnp.sum(e, axis=1, keepdims=True)
    ssoft_ref[...] = ssoft
    sq = jnp.sum(ssoft * ssoft, axis=1, keepdims=True)
    sq_ref[...] = sq
    sqt_ref[...] = sq.T

    xb = x_ref[...]
    z0_ref[...] = jnp.maximum(
        jnp.dot(xb, wc0_ref[...], preferred_element_type=jnp.float32)
        + bc0_ref[...], 0.0)
    h1_ref[...] = jnp.dot(xb, wc1_ref[...], preferred_element_type=jnp.float32)
    h2_ref[...] = jnp.dot(xb, wc2_ref[...], preferred_element_type=jnp.float32)
    h3_ref[...] = jnp.dot(xb, wc3_ref[...], preferred_element_type=jnp.float32)
    he1 = jnp.dot(xb, we1_ref[...], preferred_element_type=jnp.float32)
    hs_ref[...] = jnp.concatenate([dg * he1, ssoft], axis=1)

    stsb = jax.lax.dot_general(ssoft, ssoft, (((0,), (0,)), ((), ())),
                               preferred_element_type=jnp.float32)
    frob = jnp.sum(ab * ab).reshape(1, 1)

    @pl.when(pi == 0)
    def _():
        sts_ref[...] = stsb
        fro2_ref[...] = frob

    @pl.when(pi > 0)
    def _():
        sts_ref[...] += stsb
        fro2_ref[...] += frob


def _kc(adj_ref, ssoft_ref, sq_ref, sqt_ref, dg_ref, diag_ref, hs_ref,
        be1_ref, h1_ref, h2_ref, h3_ref, we2_ref,
        hd2_ref, t1_ref, t2_ref, t3_ref, i1_ref, i2_ref, i3_ref,
        w1_ref, w2_ref, w3_ref, cross_ref, *, R, N, H):
    pi = pl.program_id(0)
    ab = adj_ref[...]
    ssoft_f = ssoft_ref[...]
    ssoft_b = ssoft_ref[pl.ds(pi * R, R), :]
    sq_b = sq_ref[pl.ds(pi * R, R), :]
    sq_row = sqt_ref[...]

    ssm = jax.lax.dot_general(-2.0 * ssoft_b, ssoft_f,
                              (((1,), (1,)), ((), ())),
                              preferred_element_type=jnp.float32)

    big2 = jnp.dot(ab, hs_ref[...], preferred_element_type=jnp.float32)
    acc = big2[:, :H]
    p = big2[:, H:]
    cb = jnp.sum(ssoft_b * p).reshape(1, 1)

    @pl.when(pi == 0)
    def _():
        cross_ref[...] = cb

    @pl.when(pi > 0)
    def _():
        cross_ref[...] += cb

    d2 = jnp.maximum(sq_b + sq_row + ssm, 1e-9)

    cidf = jax.lax.broadcasted_iota(jnp.int32, (R, N), 1).astype(jnp.float32)
    bigi = jnp.float32(N)
    big = jnp.float32(jnp.inf)

    m1 = jnp.min(d2, axis=1, keepdims=True)
    i1f = jnp.min(jnp.where(d2 == m1, cidf, bigi), axis=1, keepdims=True)
    d2 = jnp.where(cidf == i1f, big, d2)
    m2 = jnp.min(d2, axis=1, keepdims=True)
    i2f = jnp.min(jnp.where(d2 == m2, cidf, bigi), axis=1, keepdims=True)
    d2 = jnp.where(cidf == i2f, big, d2)
    m3 = jnp.min(d2, axis=1, keepdims=True)
    i3f = jnp.min(jnp.where(d2 == m3, cidf, bigi), axis=1, keepdims=True)

    i1 = i1f.astype(jnp.int32)
    i2 = i2f.astype(jnp.int32)
    i3 = i3f.astype(jnp.int32)
    i1_ref[...] = i1
    i2_ref[...] = i2
    i3_ref[...] = i3
    v1 = jnp.exp(-jnp.sqrt(m1))
    v2 = jnp.exp(-jnp.sqrt(m2))
    v3 = jnp.exp(-jnp.sqrt(m3))
    w1_ref[...] = v1
    w2_ref[...] = v2
    w3_ref[...] = v3

    rid_b = jax.lax.broadcasted_iota(jnp.int32, (R, 1), 0) + pi * R
    for i_b, v_b, h_ref, t_ref in ((i1, v1, h1_ref, t1_ref),
                                   (i2, v2, h2_ref, t2_ref),
                                   (i3, v3, h3_ref, t3_ref)):
        off = i_b != rid_b
        dgj = jax.lax.rsqrt(1.0 + jnp.where(off, v_b, 0.0))
        t_ref[...] = dgj * h_ref[...]

    dg_b = dg_ref[pl.ds(pi * R, R), :]
    hd1_blk = hs_ref[pl.ds(pi * R, R), :H]
    acc = acc + (1.0 - diag_ref[...]) * hd1_blk
    extra1 = jnp.maximum(dg_b * acc + be1_ref[...], 0.0)
    hd2_ref[...] = dg_b * jnp.dot(extra1, we2_ref[...],
                                  preferred_element_type=jnp.float32)


def _sc_gather(t1, t2, t3, idx1, idx2, idx3):
    N, H = t1.shape
    info = plsc.get_sparse_core_info()
    nw = info.num_cores * info.num_subcores
    bpw = N // nw
    mesh = plsc.VectorSubcoreMesh(core_axis_name="c", subcore_axis_name="s")
    f32 = jnp.float32

    @functools.partial(
        pl.kernel, mesh=mesh,
        out_type=[jax.ShapeDtypeStruct((N, H), f32)] * 3,
        scratch_types=[
            pltpu.VMEM((bpw,), jnp.int32),
            pltpu.VMEM((bpw, H), f32),
            pltpu.SemaphoreType.DMA,
        ],
    )
    def g(t1_hbm, t2_hbm, t3_hbm, i1_hbm, i2_hbm, i3_hbm,
          o1_hbm, o2_hbm, o3_hbm, idx_v, rows_v, sem):
        wid = lax.axis_index("s") * info.num_cores + lax.axis_index("c")
        base = wid * bpw
        for t_hbm, i_hbm, o_hbm in ((t1_hbm, i1_hbm, o1_hbm),
                                    (t2_hbm, i2_hbm, o2_hbm),
                                    (t3_hbm, i3_hbm, o3_hbm)):
            pltpu.sync_copy(i_hbm.at[pl.ds(base, bpw)], idx_v)
            pltpu.async_copy(t_hbm.at[idx_v], rows_v, sem).wait()
            pltpu.sync_copy(rows_v, o_hbm.at[pl.ds(base, bpw)])

    return g(t1, t2, t3, idx1, idx2, idx3)


def _kd(adj_ref, hd2_ref, dg_ref, diag_ref, be2_ref, z0_ref,
        t1_ref, t2_ref, t3_ref, g1_ref, g2_ref, g3_ref,
        i1_ref, i2_ref, i3_ref, w1_ref, w2_ref, w3_ref,
        bc1_ref, bc2_ref, bc3_ref, wcls_ref, bcls_ref, sts_ref,
        cross_ref, fro2_ref, out_ref, loss_ref, *, R, N, K):
    pi = pl.program_id(0)
    ab = adj_ref[...]
    hd2 = hd2_ref[...]
    dg_b = dg_ref[pl.ds(pi * R, R), :]
    hd2_b = hd2_ref[pl.ds(pi * R, R), :]
    acc = jnp.dot(ab, hd2, preferred_element_type=jnp.float32)
    acc = acc + (1.0 - diag_ref[...]) * hd2_b
    extra2 = jnp.maximum(dg_b * acc + be2_ref[...], 0.0)

    rid_b = jax.lax.broadcasted_iota(jnp.int32, (R, 1), 0) + pi * R

    def jump(t_ref, g_ref, i_ref, w_ref, bc_ref):
        idx_b = i_ref[...]
        v_b = w_ref[...]
        coef = jnp.where(idx_b != rid_b, v_b, 0.0)
        dgj = jax.lax.rsqrt(1.0 + coef)
        return jnp.maximum(
            dgj * (t_ref[...] + coef * g_ref[...]) + bc_ref[...], 0.0)

    z1 = jump(t1_ref, g1_ref, i1_ref, w1_ref, bc1_ref)
    z2 = jump(t2_ref, g2_ref, i2_ref, w2_ref, bc2_ref)
    z3 = jump(t3_ref, g3_ref, i3_ref, w3_ref, bc3_ref)

    feat = jnp.concatenate([z0_ref[...], z1, z2, z3, extra2], axis=1)
    u = jnp.dot(feat, wcls_ref[...], preferred_element_type=jnp.float32)
    u = u + bcls_ref[...]
    mx = jnp.max(u, axis=1, keepdims=True)
    um = u - mx
    lse = jnp.log(jnp.sum(jnp.exp(um), axis=1, keepdims=True))
    out_ref[...] = um - lse

    @pl.when(pi == 0)
    def _():
        sts = sts_ref[...]
        nsts2 = jnp.sum(sts * sts)
        fro2 = fro2_ref[0, 0]
        resid = jnp.maximum(fro2 - 2.0 * cross_ref[0, 0] + nsts2, 0.0)
        pump = jnp.sqrt(resid) * jax.lax.rsqrt(fro2)
        nsts = jnp.sqrt(nsts2)
        rid = jax.lax.broadcasted_iota(jnp.int32, (K, K), 0)
        cid = jax.lax.broadcasted_iota(jnp.int32, (K, K), 1)
        eye = jnp.where(rid == cid, 1.0, 0.0)
        t = sts / nsts - eye * jax.lax.rsqrt(jnp.float32(K))
        ortho = jnp.sqrt(jnp.sum(t * t))
        loss_ref[...] = (pump + ortho).reshape(1, 1)


def kernel(x, adj, W_mlp, b_mlp, Wc, bc, We1, be1, We2, be2, att, Wcls,
           bcls):
    N = adj.shape[0]
    F = x.shape[1]
    C = W_mlp.shape[1]
    H = Wc.shape[2]
    O = Wcls.shape[1]
    R = 512 if N % 512 == 0 else N
    NB = N // R
    RC = 256 if N % 256 == 0 else N
    NBC = N // RC
    f32 = jnp.float32
    i32 = jnp.int32

    mask_att = jax.nn.softmax(att, axis=0)
    wc0 = Wc[0] * mask_att[0]
    bc0 = (bc[0] * mask_att[0]).reshape(1, H)
    bc1 = (bc[1] * mask_att[1]).reshape(1, H)
    bc2 = (bc[2] * mask_att[2]).reshape(1, H)
    bc3 = (bc[3] * mask_att[3]).reshape(1, H)
    wc1m, wc2m, wc3m = (Wc[1] * mask_att[1], Wc[2] * mask_att[2],
                        Wc[3] * mask_att[3])
    we2m = We2 * mask_att[4]
    be2m = (be2 * mask_att[4]).reshape(1, H)
    bmlp = b_mlp.reshape(1, C)
    be1r = be1.reshape(1, H)
    bclsr = bcls.reshape(1, O)

    row_blk = lambda w: pl.BlockSpec((R, w), lambda i: (i, 0))
    full = lambda a, b: pl.BlockSpec((a, b), lambda i: (0, 0))
    col_blk = pl.BlockSpec((1, R), lambda i: (0, i))
    seq = pltpu.CompilerParams(dimension_semantics=("arbitrary",))

    wmlp_aug = jnp.concatenate([W_mlp, jnp.ones((N, 1), f32)], axis=1)

    (ssoft, sq, sqt, dg, diag, z0, h1, h2, h3, hs, sts,
     fro2) = pl.pallas_call(
        functools.partial(_ka, R=R, N=N, C=C),
        grid=(NB,),
        in_specs=[row_blk(N), full(N, C + 1), full(1, C), row_blk(F),
                  full(F, H), full(F, H), full(F, H), full(F, H),
                  full(F, H), full(1, H)],
        out_specs=[row_blk(C), row_blk(1), col_blk, row_blk(1),
                   row_blk(1), row_blk(H), row_blk(H), row_blk(H),
                   row_blk(H), row_blk(2 * H), full(C, C), full(1, 1)],
        out_shape=[
            jax.ShapeDtypeStruct((N, C), f32),
            jax.ShapeDtypeStruct((N, 1), f32),
            jax.ShapeDtypeStruct((1, N), f32),
            jax.ShapeDtypeStruct((N, 1), f32),
            jax.ShapeDtypeStruct((N, 1), f32),
            jax.ShapeDtypeStruct((N, H), f32),
            jax.ShapeDtypeStruct((N, H), f32),
            jax.ShapeDtypeStruct((N, H), f32),
            jax.ShapeDtypeStruct((N, H), f32),
            jax.ShapeDtypeStruct((N, 2 * H), f32),
            jax.ShapeDtypeStruct((C, C), f32),
            jax.ShapeDtypeStruct((1, 1), f32),
        ],
        compiler_params=seq,
    )(adj, wmlp_aug, bmlp, x, wc0, wc1m, wc2m, wc3m, We1, bc0)

    row_blkc = lambda w: pl.BlockSpec((RC, w), lambda i: (i, 0))
    hd2, t1, t2, t3, i1, i2, i3, w1, w2, w3, cross = pl.pallas_call(
        functools.partial(_kc, R=RC, N=N, H=H),
        grid=(NBC,),
        in_specs=[row_blkc(N), full(N, C), full(N, 1), full(1, N),
                  full(N, 1), row_blkc(1), full(N, 2 * H), full(1, H),
                  row_blkc(H), row_blkc(H), row_blkc(H), full(H, H)],
        out_specs=[row_blkc(H), row_blkc(H), row_blkc(H), row_blkc(H),
                   row_blkc(1), row_blkc(1), row_blkc(1), row_blkc(1),
                   row_blkc(1), row_blkc(1), full(1, 1)],
        out_shape=[
            jax.ShapeDtypeStruct((N, H), f32),
            jax.ShapeDtypeStruct((N, H), f32),
            jax.ShapeDtypeStruct((N, H), f32),
            jax.ShapeDtypeStruct((N, H), f32),
            jax.ShapeDtypeStruct((N, 1), i32),
            jax.ShapeDtypeStruct((N, 1), i32),
            jax.ShapeDtypeStruct((N, 1), i32),
            jax.ShapeDtypeStruct((N, 1), f32),
            jax.ShapeDtypeStruct((N, 1), f32),
            jax.ShapeDtypeStruct((N, 1), f32),
            jax.ShapeDtypeStruct((1, 1), f32),
        ],
        compiler_params=seq,
    )(adj, ssoft, sq, sqt, dg, diag, hs, be1r, h1, h2, h3, we2m)

    g1, g2, g3 = _sc_gather(t1, t2, t3,
                            i1.reshape(N), i2.reshape(N), i3.reshape(N))

    out, loss = pl.pallas_call(
        functools.partial(_kd, R=R, N=N, K=C),
        grid=(NB,),
        in_specs=[row_blk(N), full(N, H), full(N, 1), row_blk(1),
                  full(1, H), row_blk(H), row_blk(H), row_blk(H),
                  row_blk(H), row_blk(H), row_blk(H), row_blk(H),
                  row_blk(1), row_blk(1), row_blk(1), row_blk(1),
                  row_blk(1), row_blk(1), full(1, H), full(1, H),
                  full(1, H), full(5 * H, O), full(1, O), full(C, C),
                  full(1, 1), full(1, 1)],
        out_specs=[row_blk(O), full(1, 1)],
        out_shape=[
            jax.ShapeDtypeStruct((N, O), f32),
            jax.ShapeDtypeStruct((1, 1), f32),
        ],
        compiler_params=seq,
    )(adj, hd2, dg, diag, be2m, z0, t1, t2, t3, g1, g2, g3, i1, i2, i3,
      w1, w2, w3, bc1, bc2, bc3, Wcls, bclsr, sts, cross, fro2)

    return out, loss[0, 0]


# speedup vs baseline: 1.0238x; 1.0135x over previous
import functools
import jax
import jax.numpy as jnp
from jax import lax
from jax.experimental import pallas as pl
from jax.experimental.pallas import tpu as pltpu
from jax.experimental.pallas import tpu_sc as plsc


def _ka(adj_ref, wmlp_ref, bmlp_ref, x_ref, wc0_ref, wc1_ref, wc2_ref,
        wc3_ref, we1_ref, bc0_ref,
        ssoft_ref, sq_ref, sqt_ref, dg_ref, diag_ref, z0_ref, h1_ref,
        h2_ref, h3_ref, hs_ref, sts_ref, fro2_ref, *, R, N, C):
    pi = pl.program_id(0)
    ab = adj_ref[...]
    rid = jax.lax.broadcasted_iota(jnp.int32, (R, N), 0) + pi * R
    cid = jax.lax.broadcasted_iota(jnp.int32, (R, N), 1)
    dmask = rid == cid
    diag = jnp.sum(jnp.where(dmask, ab, 0.0), axis=1, keepdims=True)

    sraw_a = jnp.dot(ab, wmlp_ref[...], preferred_element_type=jnp.float32)
    rowsum = sraw_a[:, C:C + 1]
    sraw = sraw_a[:, :C] + bmlp_ref[...]

    deg = jnp.maximum(rowsum - diag + 1.0, 1.0)
    dg = jax.lax.rsqrt(deg)
    dg_ref[...] = dg
    diag_ref[...] = diag

    m = jnp.max(sraw, axis=1, keepdims=True)
    e = jnp.exp(sraw - m)
    ssoft = e / jnp.sum(e, axis=1, keepdims=True)
    ssoft_ref[...] = ssoft
    sq = jnp.sum(ssoft * ssoft, axis=1, keepdims=True)
    sq_ref[...] = sq
    sqt_ref[...] = sq.T

    xb = x_ref[...]
    z0_ref[...] = jnp.maximum(
        jnp.dot(xb, wc0_ref[...], preferred_element_type=jnp.float32)
        + bc0_ref[...], 0.0)
    h1_ref[...] = jnp.dot(xb, wc1_ref[...], preferred_element_type=jnp.float32)
    h2_ref[...] = jnp.dot(xb, wc2_ref[...], preferred_element_type=jnp.float32)
    h3_ref[...] = jnp.dot(xb, wc3_ref[...], preferred_element_type=jnp.float32)
    he1 = jnp.dot(xb, we1_ref[...], preferred_element_type=jnp.float32)
    hs_ref[...] = jnp.concatenate([dg * he1, ssoft], axis=1)

    stsb = jax.lax.dot_general(ssoft, ssoft, (((0,), (0,)), ((), ())),
                               preferred_element_type=jnp.float32)
    frob = jnp.sum(ab * ab).reshape(1, 1)

    @pl.when(pi == 0)
    def _():
        sts_ref[...] = stsb
        fro2_ref[...] = frob

    @pl.when(pi > 0)
    def _():
        sts_ref[...] += stsb
        fro2_ref[...] += frob


def _kc(adj_ref, ssoft_ref, sq_ref, sqt_ref, dg_ref, diag_ref, hs_ref,
        be1_ref, h1_ref, h2_ref, h3_ref, we2_ref,
        hd2_ref, t1_ref, t2_ref, t3_ref, i1_ref, i2_ref, i3_ref,
        w1_ref, w2_ref, w3_ref, cross_ref, *, R, N, H):
    pi = pl.program_id(0)
    ab = adj_ref[...]
    ssoft_f = ssoft_ref[...]
    ssoft_b = ssoft_ref[pl.ds(pi * R, R), :]
    sq_b = sq_ref[pl.ds(pi * R, R), :]
    sq_row = sqt_ref[...]

    ssm = jax.lax.dot_general(-2.0 * ssoft_b, ssoft_f,
                              (((1,), (1,)), ((), ())),
                              preferred_element_type=jnp.float32)

    big2 = jnp.dot(ab, hs_ref[...], preferred_element_type=jnp.float32)
    acc = big2[:, :H]
    p = big2[:, H:]
    cb = jnp.sum(ssoft_b * p).reshape(1, 1)

    @pl.when(pi == 0)
    def _():
        cross_ref[...] = cb

    @pl.when(pi > 0)
    def _():
        cross_ref[...] += cb

    d2 = jnp.maximum(sq_b + sq_row + ssm, 1e-9)

    cidf = jax.lax.broadcasted_iota(jnp.int32, (R, N), 1).astype(jnp.float32)
    bigi = jnp.float32(N)
    big = jnp.float32(jnp.inf)

    m1 = jnp.min(d2, axis=1, keepdims=True)
    i1f = jnp.min(jnp.where(d2 == m1, cidf, bigi), axis=1, keepdims=True)
    d2 = jnp.where(cidf == i1f, big, d2)
    m2 = jnp.min(d2, axis=1, keepdims=True)
    i2f = jnp.min(jnp.where(d2 == m2, cidf, bigi), axis=1, keepdims=True)
    d2 = jnp.where(cidf == i2f, big, d2)
    m3 = jnp.min(d2, axis=1, keepdims=True)
    i3f = jnp.min(jnp.where(d2 == m3, cidf, bigi), axis=1, keepdims=True)

    i1 = i1f.astype(jnp.int32)
    i2 = i2f.astype(jnp.int32)
    i3 = i3f.astype(jnp.int32)
    i1_ref[...] = i1
    i2_ref[...] = i2
    i3_ref[...] = i3
    v1 = jnp.exp(-jnp.sqrt(m1))
    v2 = jnp.exp(-jnp.sqrt(m2))
    v3 = jnp.exp(-jnp.sqrt(m3))
    w1_ref[...] = v1
    w2_ref[...] = v2
    w3_ref[...] = v3

    rid_b = jax.lax.broadcasted_iota(jnp.int32, (R, 1), 0) + pi * R
    for i_b, v_b, h_ref, t_ref in ((i1, v1, h1_ref, t1_ref),
                                   (i2, v2, h2_ref, t2_ref),
                                   (i3, v3, h3_ref, t3_ref)):
        off = i_b != rid_b
        dgj = jax.lax.rsqrt(1.0 + jnp.where(off, v_b, 0.0))
        t_ref[...] = dgj * h_ref[...]

    dg_b = dg_ref[pl.ds(pi * R, R), :]
    hd1_blk = hs_ref[pl.ds(pi * R, R), :H]
    acc = acc + (1.0 - diag_ref[...]) * hd1_blk
    extra1 = jnp.maximum(dg_b * acc + be1_ref[...], 0.0)
    hd2_ref[...] = dg_b * jnp.dot(extra1, we2_ref[...],
                                  preferred_element_type=jnp.float32)


def _sc_gather(t1, t2, t3, idx1, idx2, idx3):
    N, H = t1.shape
    dt = t1.dtype
    info = plsc.get_sparse_core_info()
    nw = info.num_cores * info.num_subcores
    bpw = N // nw
    mesh = plsc.VectorSubcoreMesh(core_axis_name="c", subcore_axis_name="s")

    @functools.partial(
        pl.kernel, mesh=mesh,
        out_type=[jax.ShapeDtypeStruct((N, H), dt)] * 3,
        scratch_types=[
            pltpu.VMEM((bpw,), jnp.int32),
            pltpu.VMEM((bpw,), jnp.int32),
            pltpu.VMEM((bpw,), jnp.int32),
            pltpu.VMEM((bpw, H), dt),
            pltpu.VMEM((bpw, H), dt),
            pltpu.VMEM((bpw, H), dt),
            pltpu.SemaphoreType.DMA,
        ],
    )
    def g(t1_hbm, t2_hbm, t3_hbm, i1_hbm, i2_hbm, i3_hbm,
          o1_hbm, o2_hbm, o3_hbm, i1_v, i2_v, i3_v, r1_v, r2_v, r3_v,
          sem):
        wid = lax.axis_index("s") * info.num_cores + lax.axis_index("c")
        base = wid * bpw
        pltpu.sync_copy(i1_hbm.at[pl.ds(base, bpw)], i1_v)
        pltpu.sync_copy(i2_hbm.at[pl.ds(base, bpw)], i2_v)
        pltpu.sync_copy(i3_hbm.at[pl.ds(base, bpw)], i3_v)
        c1 = pltpu.async_copy(t1_hbm.at[i1_v], r1_v, sem)
        c2 = pltpu.async_copy(t2_hbm.at[i2_v], r2_v, sem)
        c3 = pltpu.async_copy(t3_hbm.at[i3_v], r3_v, sem)
        c1.wait()
        pltpu.sync_copy(r1_v, o1_hbm.at[pl.ds(base, bpw)])
        c2.wait()
        pltpu.sync_copy(r2_v, o2_hbm.at[pl.ds(base, bpw)])
        c3.wait()
        pltpu.sync_copy(r3_v, o3_hbm.at[pl.ds(base, bpw)])

    return g(t1, t2, t3, idx1, idx2, idx3)


def _kd(adj_ref, hd2_ref, dg_ref, diag_ref, be2_ref, z0_ref,
        h1_ref, h2_ref, h3_ref, g1_ref, g2_ref, g3_ref,
        i1_ref, i2_ref, i3_ref, w1_ref, w2_ref, w3_ref,
        bc1_ref, bc2_ref, bc3_ref, wcls_ref, bcls_ref, sts_ref,
        cross_ref, fro2_ref, out_ref, loss_ref, *, R, N, K):
    pi = pl.program_id(0)
    ab = adj_ref[...]
    hd2 = hd2_ref[...]
    dg_b = dg_ref[pl.ds(pi * R, R), :]
    hd2_b = hd2_ref[pl.ds(pi * R, R), :]
    acc = jnp.dot(ab, hd2, preferred_element_type=jnp.float32)
    acc = acc + (1.0 - diag_ref[...]) * hd2_b
    extra2 = jnp.maximum(dg_b * acc + be2_ref[...], 0.0)

    rid_b = jax.lax.broadcasted_iota(jnp.int32, (R, 1), 0) + pi * R

    def jump(h_ref, g_ref, i_ref, w_ref, bc_ref):
        idx_b = i_ref[...]
        v_b = w_ref[...]
        coef = jnp.where(idx_b != rid_b, v_b, 0.0)
        dgj = jax.lax.rsqrt(1.0 + coef)
        g_b = g_ref[...].astype(jnp.float32)
        return jnp.maximum(
            dgj * (dgj * h_ref[...] + coef * g_b) + bc_ref[...], 0.0)

    z1 = jump(h1_ref, g1_ref, i1_ref, w1_ref, bc1_ref)
    z2 = jump(h2_ref, g2_ref, i2_ref, w2_ref, bc2_ref)
    z3 = jump(h3_ref, g3_ref, i3_ref, w3_ref, bc3_ref)

    feat = jnp.concatenate([z0_ref[...], z1, z2, z3, extra2], axis=1)
    u = jnp.dot(feat, wcls_ref[...], preferred_element_type=jnp.float32)
    u = u + bcls_ref[...]
    mx = jnp.max(u, axis=1, keepdims=True)
    um = u - mx
    lse = jnp.log(jnp.sum(jnp.exp(um), axis=1, keepdims=True))
    out_ref[...] = um - lse

    @pl.when(pi == 0)
    def _():
        sts = sts_ref[...]
        nsts2 = jnp.sum(sts * sts)
        fro2 = fro2_ref[0, 0]
        resid = jnp.maximum(fro2 - 2.0 * cross_ref[0, 0] + nsts2, 0.0)
        pump = jnp.sqrt(resid) * jax.lax.rsqrt(fro2)
        nsts = jnp.sqrt(nsts2)
        rid = jax.lax.broadcasted_iota(jnp.int32, (K, K), 0)
        cid = jax.lax.broadcasted_iota(jnp.int32, (K, K), 1)
        eye = jnp.where(rid == cid, 1.0, 0.0)
        t = sts / nsts - eye * jax.lax.rsqrt(jnp.float32(K))
        ortho = jnp.sqrt(jnp.sum(t * t))
        loss_ref[...] = (pump + ortho).reshape(1, 1)


def kernel(x, adj, W_mlp, b_mlp, Wc, bc, We1, be1, We2, be2, att, Wcls,
           bcls):
    N = adj.shape[0]
    F = x.shape[1]
    C = W_mlp.shape[1]
    H = Wc.shape[2]
    O = Wcls.shape[1]
    R = 512 if N % 512 == 0 else N
    NB = N // R
    RC = R
    NBC = N // RC
    f32 = jnp.float32
    i32 = jnp.int32

    mask_att = jax.nn.softmax(att, axis=0)
    wc0 = Wc[0] * mask_att[0]
    bc0 = (bc[0] * mask_att[0]).reshape(1, H)
    bc1 = (bc[1] * mask_att[1]).reshape(1, H)
    bc2 = (bc[2] * mask_att[2]).reshape(1, H)
    bc3 = (bc[3] * mask_att[3]).reshape(1, H)
    wc1m, wc2m, wc3m = (Wc[1] * mask_att[1], Wc[2] * mask_att[2],
                        Wc[3] * mask_att[3])
    we2m = We2 * mask_att[4]
    be2m = (be2 * mask_att[4]).reshape(1, H)
    bmlp = b_mlp.reshape(1, C)
    be1r = be1.reshape(1, H)
    bclsr = bcls.reshape(1, O)

    row_blk = lambda w: pl.BlockSpec((R, w), lambda i: (i, 0))
    full = lambda a, b: pl.BlockSpec((a, b), lambda i: (0, 0))
    col_blk = pl.BlockSpec((1, R), lambda i: (0, i))
    seq = pltpu.CompilerParams(dimension_semantics=("arbitrary",))

    wmlp_aug = jnp.concatenate([W_mlp, jnp.ones((N, 1), f32)], axis=1)

    (ssoft, sq, sqt, dg, diag, z0, h1, h2, h3, hs, sts,
     fro2) = pl.pallas_call(
        functools.partial(_ka, R=R, N=N, C=C),
        grid=(NB,),
        in_specs=[row_blk(N), full(N, C + 1), full(1, C), row_blk(F),
                  full(F, H), full(F, H), full(F, H), full(F, H),
                  full(F, H), full(1, H)],
        out_specs=[row_blk(C), row_blk(1), col_blk, row_blk(1),
                   row_blk(1), row_blk(H), row_blk(H), row_blk(H),
                   row_blk(H), row_blk(2 * H), full(C, C), full(1, 1)],
        out_shape=[
            jax.ShapeDtypeStruct((N, C), f32),
            jax.ShapeDtypeStruct((N, 1), f32),
            jax.ShapeDtypeStruct((1, N), f32),
            jax.ShapeDtypeStruct((N, 1), f32),
            jax.ShapeDtypeStruct((N, 1), f32),
            jax.ShapeDtypeStruct((N, H), f32),
            jax.ShapeDtypeStruct((N, H), f32),
            jax.ShapeDtypeStruct((N, H), f32),
            jax.ShapeDtypeStruct((N, H), f32),
            jax.ShapeDtypeStruct((N, 2 * H), f32),
            jax.ShapeDtypeStruct((C, C), f32),
            jax.ShapeDtypeStruct((1, 1), f32),
        ],
        compiler_params=seq,
    )(adj, wmlp_aug, bmlp, x, wc0, wc1m, wc2m, wc3m, We1, bc0)

    row_blkc = lambda w: pl.BlockSpec((RC, w), lambda i: (i, 0))
    hd2, t1, t2, t3, i1, i2, i3, w1, w2, w3, cross = pl.pallas_call(
        functools.partial(_kc, R=RC, N=N, H=H),
        grid=(NBC,),
        in_specs=[row_blkc(N), full(N, C), full(N, 1), full(1, N),
                  full(N, 1), row_blkc(1), full(N, 2 * H), full(1, H),
                  row_blkc(H), row_blkc(H), row_blkc(H), full(H, H)],
        out_specs=[row_blkc(H), row_blkc(H), row_blkc(H), row_blkc(H),
                   row_blkc(1), row_blkc(1), row_blkc(1), row_blkc(1),
                   row_blkc(1), row_blkc(1), full(1, 1)],
        out_shape=[
            jax.ShapeDtypeStruct((N, H), f32),
            jax.ShapeDtypeStruct((N, H), f32),
            jax.ShapeDtypeStruct((N, H), f32),
            jax.ShapeDtypeStruct((N, H), f32),
            jax.ShapeDtypeStruct((N, 1), i32),
            jax.ShapeDtypeStruct((N, 1), i32),
            jax.ShapeDtypeStruct((N, 1), i32),
            jax.ShapeDtypeStruct((N, 1), f32),
            jax.ShapeDtypeStruct((N, 1), f32),
            jax.ShapeDtypeStruct((N, 1), f32),
            jax.ShapeDtypeStruct((1, 1), f32),
        ],
        compiler_params=seq,
    )(adj, ssoft, sq, sqt, dg, diag, hs, be1r, h1, h2, h3, we2m)

    g1, g2, g3 = _sc_gather(t1, t2, t3,
                            i1.reshape(N), i2.reshape(N), i3.reshape(N))

    out, loss = pl.pallas_call(
        functools.partial(_kd, R=R, N=N, K=C),
        grid=(NB,),
        in_specs=[row_blk(N), full(N, H), full(N, 1), row_blk(1),
                  full(1, H), row_blk(H), row_blk(H), row_blk(H),
                  row_blk(H), row_blk(H), row_blk(H), row_blk(H),
                  row_blk(1), row_blk(1), row_blk(1), row_blk(1),
                  row_blk(1), row_blk(1), full(1, H), full(1, H),
                  full(1, H), full(5 * H, O), full(1, O), full(C, C),
                  full(1, 1), full(1, 1)],
        out_specs=[row_blk(O), full(1, 1)],
        out_shape=[
            jax.ShapeDtypeStruct((N, O), f32),
            jax.ShapeDtypeStruct((1, 1), f32),
        ],
        compiler_params=seq,
    )(adj, hd2, dg, diag, be2m, z0, h1, h2, h3, g1, g2, g3, i1, i2, i3,
      w1, w2, w3, bc1, bc2, bc3, Wcls, bclsr, sts, cross, fro2)

    return out, loss[0, 0]


# speedup vs baseline: 1.0471x; 1.0227x over previous
import functools
import jax
import jax.numpy as jnp
from jax import lax
from jax.experimental import pallas as pl
from jax.experimental.pallas import tpu as pltpu
from jax.experimental.pallas import tpu_sc as plsc


def _ka(adj_ref, wmlp_ref, bmlp_ref, x_ref, we1_ref,
        sq_ref, sqt_ref, dg_ref, diag_ref, hs_ref, sts_ref, fro2_ref,
        *, R, N, C):
    pi = pl.program_id(0)
    ab = adj_ref[...]
    rid = jax.lax.broadcasted_iota(jnp.int32, (R, N), 0) + pi * R
    cid = jax.lax.broadcasted_iota(jnp.int32, (R, N), 1)
    dmask = rid == cid
    diag = jnp.sum(jnp.where(dmask, ab, 0.0), axis=1, keepdims=True)

    sraw_a = jnp.dot(ab, wmlp_ref[...], preferred_element_type=jnp.float32)
    rowsum = sraw_a[:, C:C + 1]
    sraw = sraw_a[:, :C] + bmlp_ref[...]

    deg = jnp.maximum(rowsum - diag + 1.0, 1.0)
    dg = jax.lax.rsqrt(deg)
    dg_ref[...] = dg
    diag_ref[...] = diag

    m = jnp.max(sraw, axis=1, keepdims=True)
    e = jnp.exp(sraw - m)
    ssoft = e / jnp.sum(e, axis=1, keepdims=True)
    sq = jnp.sum(ssoft * ssoft, axis=1, keepdims=True)
    sq_ref[...] = sq
    sqt_ref[...] = sq.T

    xb = x_ref[...]
    he1 = jnp.dot(xb, we1_ref[...], preferred_element_type=jnp.float32)
    hs_ref[...] = jnp.concatenate([dg * he1, ssoft], axis=1)

    stsb = jax.lax.dot_general(ssoft, ssoft, (((0,), (0,)), ((), ())),
                               preferred_element_type=jnp.float32)
    frob = jnp.sum(ab * ab).reshape(1, 1)

    @pl.when(pi == 0)
    def _():
        sts_ref[...] = stsb
        fro2_ref[...] = frob

    @pl.when(pi > 0)
    def _():
        sts_ref[...] += stsb
        fro2_ref[...] += frob


def _kc(adj_ref, sq_ref, sqt_ref, dg_ref, diag_ref, hs_ref,
        be1_ref, x_ref, wc1_ref, wc2_ref, wc3_ref, we2_ref,
        hd2_ref, t1_ref, t2_ref, t3_ref, i1_ref, i2_ref, i3_ref,
        w1_ref, w2_ref, w3_ref, cross_ref, *, R, N, H):
    pi = pl.program_id(0)
    ab = adj_ref[...]
    ssoft_f = hs_ref[:, H:]
    ssoft_b = hs_ref[pl.ds(pi * R, R), H:]
    sq_b = sq_ref[pl.ds(pi * R, R), :]
    sq_row = sqt_ref[...]

    ssm = jax.lax.dot_general(-2.0 * ssoft_b, ssoft_f,
                              (((1,), (1,)), ((), ())),
                              preferred_element_type=jnp.float32)

    big2 = jnp.dot(ab, hs_ref[...], preferred_element_type=jnp.float32)
    acc = big2[:, :H]
    p = big2[:, H:]
    cb = jnp.sum(ssoft_b * p).reshape(1, 1)

    @pl.when(pi == 0)
    def _():
        cross_ref[...] = cb

    @pl.when(pi > 0)
    def _():
        cross_ref[...] += cb

    d2 = jnp.maximum(sq_b + sq_row + ssm, 1e-9)

    cidf = jax.lax.broadcasted_iota(jnp.int32, (R, N), 1).astype(jnp.float32)
    bigi = jnp.float32(N)
    big = jnp.float32(jnp.inf)

    m1 = jnp.min(d2, axis=1, keepdims=True)
    i1f = jnp.min(jnp.where(d2 == m1, cidf, bigi), axis=1, keepdims=True)
    d2 = jnp.where(cidf == i1f, big, d2)
    m2 = jnp.min(d2, axis=1, keepdims=True)
    i2f = jnp.min(jnp.where(d2 == m2, cidf, bigi), axis=1, keepdims=True)
    d2 = jnp.where(cidf == i2f, big, d2)
    m3 = jnp.min(d2, axis=1, keepdims=True)
    i3f = jnp.min(jnp.where(d2 == m3, cidf, bigi), axis=1, keepdims=True)

    i1 = i1f.astype(jnp.int32)
    i2 = i2f.astype(jnp.int32)
    i3 = i3f.astype(jnp.int32)
    i1_ref[...] = i1
    i2_ref[...] = i2
    i3_ref[...] = i3
    v1 = jnp.exp(-jnp.sqrt(m1))
    v2 = jnp.exp(-jnp.sqrt(m2))
    v3 = jnp.exp(-jnp.sqrt(m3))
    w1_ref[...] = v1
    w2_ref[...] = v2
    w3_ref[...] = v3

    rid_b = jax.lax.broadcasted_iota(jnp.int32, (R, 1), 0) + pi * R
    xb = x_ref[...]
    for i_b, v_b, wc_ref, t_ref in ((i1, v1, wc1_ref, t1_ref),
                                    (i2, v2, wc2_ref, t2_ref),
                                    (i3, v3, wc3_ref, t3_ref)):
        off = i_b != rid_b
        dgj = jax.lax.rsqrt(1.0 + jnp.where(off, v_b, 0.0))
        h = jnp.dot(xb, wc_ref[...], preferred_element_type=jnp.float32)
        t_ref[...] = dgj * h

    dg_b = dg_ref[pl.ds(pi * R, R), :]
    hd1_blk = hs_ref[pl.ds(pi * R, R), :H]
    acc = acc + (1.0 - diag_ref[...]) * hd1_blk
    extra1 = jnp.maximum(dg_b * acc + be1_ref[...], 0.0)
    hd2_ref[...] = dg_b * jnp.dot(extra1, we2_ref[...],
                                  preferred_element_type=jnp.float32)


def _sc_gather(t1, t2, t3, idx1, idx2, idx3):
    N, H = t1.shape
    dt = t1.dtype
    info = plsc.get_sparse_core_info()
    nw = info.num_cores * info.num_subcores
    bpw = N // nw
    mesh = plsc.VectorSubcoreMesh(core_axis_name="c", subcore_axis_name="s")

    @functools.partial(
        pl.kernel, mesh=mesh,
        out_type=[jax.ShapeDtypeStruct((N, H), dt)] * 3,
        scratch_types=[
            pltpu.VMEM((bpw,), jnp.int32),
            pltpu.VMEM((bpw,), jnp.int32),
            pltpu.VMEM((bpw,), jnp.int32),
            pltpu.VMEM((bpw, H), dt),
            pltpu.VMEM((bpw, H), dt),
            pltpu.VMEM((bpw, H), dt),
            pltpu.SemaphoreType.DMA,
        ],
    )
    def g(t1_hbm, t2_hbm, t3_hbm, i1_hbm, i2_hbm, i3_hbm,
          o1_hbm, o2_hbm, o3_hbm, i1_v, i2_v, i3_v, r1_v, r2_v, r3_v,
          sem):
        wid = lax.axis_index("s") * info.num_cores + lax.axis_index("c")
        base = wid * bpw
        pltpu.sync_copy(i1_hbm.at[pl.ds(base, bpw)], i1_v)
        pltpu.sync_copy(i2_hbm.at[pl.ds(base, bpw)], i2_v)
        pltpu.sync_copy(i3_hbm.at[pl.ds(base, bpw)], i3_v)
        c1 = pltpu.async_copy(t1_hbm.at[i1_v], r1_v, sem)
        c2 = pltpu.async_copy(t2_hbm.at[i2_v], r2_v, sem)
        c3 = pltpu.async_copy(t3_hbm.at[i3_v], r3_v, sem)
        c1.wait()
        pltpu.sync_copy(r1_v, o1_hbm.at[pl.ds(base, bpw)])
        c2.wait()
        pltpu.sync_copy(r2_v, o2_hbm.at[pl.ds(base, bpw)])
        c3.wait()
        pltpu.sync_copy(r3_v, o3_hbm.at[pl.ds(base, bpw)])

    return g(t1, t2, t3, idx1, idx2, idx3)


def _kd(adj_ref, hd2_ref, dg_ref, diag_ref, be2_ref, x_ref, wc0_ref,
        bc0_ref, t1_ref, t2_ref, t3_ref, g1_ref, g2_ref, g3_ref,
        i1_ref, i2_ref, i3_ref, w1_ref, w2_ref, w3_ref,
        bc1_ref, bc2_ref, bc3_ref, wcls_ref, bcls_ref, sts_ref,
        cross_ref, fro2_ref, out_ref, loss_ref, *, R, N, K):
    pi = pl.program_id(0)
    ab = adj_ref[...]
    hd2 = hd2_ref[...]
    dg_b = dg_ref[pl.ds(pi * R, R), :]
    hd2_b = hd2_ref[pl.ds(pi * R, R), :]
    acc = jnp.dot(ab, hd2, preferred_element_type=jnp.float32)
    acc = acc + (1.0 - diag_ref[...]) * hd2_b
    extra2 = jnp.maximum(dg_b * acc + be2_ref[...], 0.0)

    z0 = jnp.maximum(
        jnp.dot(x_ref[...], wc0_ref[...], preferred_element_type=jnp.float32)
        + bc0_ref[...], 0.0)

    rid_b = jax.lax.broadcasted_iota(jnp.int32, (R, 1), 0) + pi * R

    def jump(t_ref, g_ref, i_ref, w_ref, bc_ref):
        idx_b = i_ref[...]
        v_b = w_ref[...]
        coef = jnp.where(idx_b != rid_b, v_b, 0.0)
        dgj = jax.lax.rsqrt(1.0 + coef)
        return jnp.maximum(
            dgj * (t_ref[...] + coef * g_ref[...]) + bc_ref[...], 0.0)

    z1 = jump(t1_ref, g1_ref, i1_ref, w1_ref, bc1_ref)
    z2 = jump(t2_ref, g2_ref, i2_ref, w2_ref, bc2_ref)
    z3 = jump(t3_ref, g3_ref, i3_ref, w3_ref, bc3_ref)

    feat = jnp.concatenate([z0, z1, z2, z3, extra2], axis=1)
    u = jnp.dot(feat, wcls_ref[...], preferred_element_type=jnp.float32)
    u = u + bcls_ref[...]
    mx = jnp.max(u, axis=1, keepdims=True)
    um = u - mx
    lse = jnp.log(jnp.sum(jnp.exp(um), axis=1, keepdims=True))
    out_ref[...] = um - lse

    @pl.when(pi == 0)
    def _():
        sts = sts_ref[...]
        nsts2 = jnp.sum(sts * sts)
        fro2 = fro2_ref[0, 0]
        resid = jnp.maximum(fro2 - 2.0 * cross_ref[0, 0] + nsts2, 0.0)
        pump = jnp.sqrt(resid) * jax.lax.rsqrt(fro2)
        nsts = jnp.sqrt(nsts2)
        rid = jax.lax.broadcasted_iota(jnp.int32, (K, K), 0)
        cid = jax.lax.broadcasted_iota(jnp.int32, (K, K), 1)
        eye = jnp.where(rid == cid, 1.0, 0.0)
        t = sts / nsts - eye * jax.lax.rsqrt(jnp.float32(K))
        ortho = jnp.sqrt(jnp.sum(t * t))
        loss_ref[...] = (pump + ortho).reshape(1, 1)


def kernel(x, adj, W_mlp, b_mlp, Wc, bc, We1, be1, We2, be2, att, Wcls,
           bcls):
    N = adj.shape[0]
    F = x.shape[1]
    C = W_mlp.shape[1]
    H = Wc.shape[2]
    O = Wcls.shape[1]
    R = 512 if N % 512 == 0 else N
    NB = N // R
    RC = R
    NBC = N // RC
    f32 = jnp.float32
    i32 = jnp.int32

    mask_att = jax.nn.softmax(att, axis=0)
    wc0 = Wc[0] * mask_att[0]
    bc0 = (bc[0] * mask_att[0]).reshape(1, H)
    bc1 = (bc[1] * mask_att[1]).reshape(1, H)
    bc2 = (bc[2] * mask_att[2]).reshape(1, H)
    bc3 = (bc[3] * mask_att[3]).reshape(1, H)
    wc1m, wc2m, wc3m = (Wc[1] * mask_att[1], Wc[2] * mask_att[2],
                        Wc[3] * mask_att[3])
    we2m = We2 * mask_att[4]
    be2m = (be2 * mask_att[4]).reshape(1, H)
    bmlp = b_mlp.reshape(1, C)
    be1r = be1.reshape(1, H)
    bclsr = bcls.reshape(1, O)

    row_blk = lambda w: pl.BlockSpec((R, w), lambda i: (i, 0))
    full = lambda a, b: pl.BlockSpec((a, b), lambda i: (0, 0))
    col_blk = pl.BlockSpec((1, R), lambda i: (0, i))
    seq = pltpu.CompilerParams(dimension_semantics=("arbitrary",))

    wmlp_aug = jnp.concatenate([W_mlp, jnp.ones((N, 1), f32)], axis=1)

    sq, sqt, dg, diag, hs, sts, fro2 = pl.pallas_call(
        functools.partial(_ka, R=R, N=N, C=C),
        grid=(NB,),
        in_specs=[row_blk(N), full(N, C + 1), full(1, C), row_blk(F),
                  full(F, H)],
        out_specs=[row_blk(1), col_blk, row_blk(1), row_blk(1),
                   row_blk(2 * H), full(C, C), full(1, 1)],
        out_shape=[
            jax.ShapeDtypeStruct((N, 1), f32),
            jax.ShapeDtypeStruct((1, N), f32),
            jax.ShapeDtypeStruct((N, 1), f32),
            jax.ShapeDtypeStruct((N, 1), f32),
            jax.ShapeDtypeStruct((N, 2 * H), f32),
            jax.ShapeDtypeStruct((C, C), f32),
            jax.ShapeDtypeStruct((1, 1), f32),
        ],
        compiler_params=seq,
    )(adj, wmlp_aug, bmlp, x, We1)

    row_blkc = lambda w: pl.BlockSpec((RC, w), lambda i: (i, 0))
    hd2, t1, t2, t3, i1, i2, i3, w1, w2, w3, cross = pl.pallas_call(
        functools.partial(_kc, R=RC, N=N, H=H),
        grid=(NBC,),
        in_specs=[row_blkc(N), full(N, 1), full(1, N),
                  full(N, 1), row_blkc(1), full(N, 2 * H), full(1, H),
                  row_blkc(F), full(F, H), full(F, H), full(F, H),
                  full(H, H)],
        out_specs=[row_blkc(H), row_blkc(H), row_blkc(H), row_blkc(H),
                   row_blkc(1), row_blkc(1), row_blkc(1), row_blkc(1),
                   row_blkc(1), row_blkc(1), full(1, 1)],
        out_shape=[
            jax.ShapeDtypeStruct((N, H), f32),
            jax.ShapeDtypeStruct((N, H), f32),
            jax.ShapeDtypeStruct((N, H), f32),
            jax.ShapeDtypeStruct((N, H), f32),
            jax.ShapeDtypeStruct((N, 1), i32),
            jax.ShapeDtypeStruct((N, 1), i32),
            jax.ShapeDtypeStruct((N, 1), i32),
            jax.ShapeDtypeStruct((N, 1), f32),
            jax.ShapeDtypeStruct((N, 1), f32),
            jax.ShapeDtypeStruct((N, 1), f32),
            jax.ShapeDtypeStruct((1, 1), f32),
        ],
        compiler_params=seq,
    )(adj, sq, sqt, dg, diag, hs, be1r, x, wc1m, wc2m, wc3m, we2m)

    g1, g2, g3 = _sc_gather(t1, t2, t3,
                            i1.reshape(N), i2.reshape(N), i3.reshape(N))

    out, loss = pl.pallas_call(
        functools.partial(_kd, R=R, N=N, K=C),
        grid=(NB,),
        in_specs=[row_blk(N), full(N, H), full(N, 1), row_blk(1),
                  full(1, H), row_blk(F), full(F, H), full(1, H),
                  row_blk(H), row_blk(H), row_blk(H),
                  row_blk(H), row_blk(H), row_blk(H),
                  row_blk(1), row_blk(1), row_blk(1), row_blk(1),
                  row_blk(1), row_blk(1), full(1, H), full(1, H),
                  full(1, H), full(5 * H, O), full(1, O), full(C, C),
                  full(1, 1), full(1, 1)],
        out_specs=[row_blk(O), full(1, 1)],
        out_shape=[
            jax.ShapeDtypeStruct((N, O), f32),
            jax.ShapeDtypeStruct((1, 1), f32),
        ],
        compiler_params=seq,
    )(adj, hd2, dg, diag, be2m, x, wc0, bc0, t1, t2, t3, g1, g2, g3,
      i1, i2, i3, w1, w2, w3, bc1, bc2, bc3, Wcls, bclsr, sts, cross,
      fro2)

    return out, loss[0, 0]
